# baseline (speedup 1.0000x reference)
import functools

import jax
import jax.numpy as jnp
from jax import lax
from jax.experimental import pallas as pl
from jax.experimental.pallas import tpu as pltpu
from jax.experimental.pallas import tpu_sc as plsc

_NC = 2
_NS = 16
_NW = _NC * _NS
_CHUNK = 120
_LANES = 16
_FRAC0 = 0.60


def _node_pad(n_nodes: int) -> int:
  return -(-(n_nodes + 8) // (_NS * 8)) * (_NS * 8)


def _make_agg(n_nodes: int, d: int, n0: int, n1: int, with_counts: bool):
  assert min(n0, n1) >= 4
  n_pad = _node_pad(n_nodes)
  rpt = n_pad // _NS

  mesh = plsc.VectorSubcoreMesh(core_axis_name="c", subcore_axis_name="s")

  out_type = [jax.ShapeDtypeStruct((_NC * n_pad, d), jnp.float32)]
  if with_counts:
    out_type.append(jax.ShapeDtypeStruct((_NC * n_pad, d), jnp.float32))
  scratch = [
      pltpu.VMEM((4, 2, _CHUNK), jnp.int32),
      pltpu.VMEM((3, _CHUNK, d), jnp.float32),
      pltpu.VMEM_SHARED((n_pad, d), jnp.float32),
      pltpu.SemaphoreType.DMA,
      pltpu.SemaphoreType.DMA,
      pltpu.SemaphoreType.DMA,
  ]

  @functools.partial(pl.kernel, mesh=mesh, out_type=out_type,
                     scratch_types=scratch)
  def agg(table_hbm, ids_hbm, *refs):
    if with_counts:
      (out_hbm, cnt_hbm, idsw, rows_v, acc_sh, gsem, ssem, isem) = refs
    else:
      (out_hbm, idsw, rows_v, acc_sh, gsem, ssem, isem) = refs
      cnt_hbm = None

    cid = lax.axis_index("c")
    sid = lax.axis_index("s")
    w = cid * _NS + sid
    nc = jnp.where(cid == 0, n0, n1)
    r0 = pl.multiple_of(sid * rpt, 8)
    ro = pl.multiple_of(cid * n_pad + sid * rpt, 8)

    def fill_slot(slot, val):
      def _f(i, _):
        rows_v[slot, i // (d // _LANES),
               pl.ds((i % (d // _LANES)) * _LANES, _LANES)] = (
                   jnp.full((_LANES,), val, jnp.float32))
        return 0
      lax.fori_loop(0, _CHUNK * (d // _LANES), _f, 0)

    def zero_acc_slice(slot):
      done = 0
      while done < rpt:
        step = min(_CHUNK, rpt - done)
        pltpu.sync_copy(rows_v.at[slot, pl.ds(0, step)],
                        acc_sh.at[pl.ds(r0 + done, step)])
        done += step

    def drain_gather():
      pltpu.make_async_copy(table_hbm.at[pl.ds(0, _CHUNK)], rows_v.at[0],
                            gsem).wait()

    def drain_scatter():
      pltpu.make_async_copy(rows_v.at[0], acc_sh.at[pl.ds(0, _CHUNK)],
                            ssem).wait()

    def drain_ids():
      pltpu.make_async_copy(ids_hbm.at[w, 0], idsw.at[0], isem).wait()

    def sync_ids(c, slot):
      pltpu.sync_copy(ids_hbm.at[w, c], idsw.at[slot])

    def issue_ids(c, slot):
      pltpu.async_copy(ids_hbm.at[w, c], idsw.at[slot], isem)

    def issue_gather(slot_ids, slot_rows):
      pltpu.async_copy(table_hbm.at[idsw.at[slot_ids, 0]],
                       rows_v.at[slot_rows], gsem)

    def issue_scatter(slot_ids, slot_rows):
      pltpu.async_copy(rows_v.at[slot_rows], acc_sh.at[idsw.at[slot_ids, 1]],
                       ssem, add=True)

    def counts_pass():
      sync_ids(0, 0)
      issue_ids(1, 1)
      issue_ids(2, 2)
      issue_ids(3, 3)

      def _b(c, _):
        @pl.when(c >= 2)
        def _():
          drain_scatter()
          @pl.when(c + 2 < nc)
          def _():
            issue_ids(c + 2, lax.rem(c + 2, 4))
        @pl.when(c >= 1)
        def _():
          drain_ids()
        issue_scatter(lax.rem(c, 4), 0)
        return 0
      lax.fori_loop(0, nc, _b, 0)
      drain_scatter()
      drain_scatter()

    def feature_pass():
      sync_ids(0, 0)
      issue_gather(0, 0)
      issue_ids(1, 1)
      drain_ids()
      issue_gather(1, 1)
      issue_ids(2, 2)
      issue_ids(3, 3)

      def _b(c, _):
        drain_gather()
        @pl.when(c >= 1)
        def _():
          drain_scatter()
          @pl.when(c + 3 < nc)
          def _():
            issue_ids(c + 3, lax.rem(c + 3, 4))
        @pl.when(c + 2 < nc)
        def _():
          drain_ids()
          issue_gather(lax.rem(c + 2, 4), lax.rem(c + 2, 3))
        issue_scatter(lax.rem(c, 4), lax.rem(c, 3))
        return 0
      lax.fori_loop(0, nc, _b, 0)
      drain_scatter()

    def copy_out(dst_hbm_ref):
      pltpu.sync_copy(acc_sh.at[pl.ds(r0, rpt)], dst_hbm_ref.at[pl.ds(ro, rpt)])

    if with_counts:
      fill_slot(0, 1.0)
      fill_slot(1, 0.0)
      zero_acc_slice(1)
      plsc.subcore_barrier()
      counts_pass()
      plsc.subcore_barrier()
      copy_out(cnt_hbm)
      zero_acc_slice(1)
    else:
      fill_slot(1, 0.0)
      zero_acc_slice(1)
    plsc.subcore_barrier()
    feature_pass()
    plsc.subcore_barrier()
    copy_out(out_hbm)

  return agg


def _dense_layer(sums, cnts, x, w_l, b_l, w_r, apply_relu):
  n, d = x.shape
  n_pad = _node_pad(n)

  def body(s_ref, c_ref, x_ref, wl_ref, bl_ref, wr_ref, o_ref):
    s = s_ref[...]
    s = s[:n] + s[n_pad:n_pad + n]
    c = c_ref[...]
    c = c[:n, 0:1] + c[n_pad:n_pad + n, 0:1]
    mean = s * (1.0 / jnp.maximum(c, 1.0))
    acc = jnp.dot(mean, wl_ref[...], preferred_element_type=jnp.float32)
    acc = acc + jnp.dot(x_ref[...], wr_ref[...],
                        preferred_element_type=jnp.float32)
    acc = acc + bl_ref[...]
    if apply_relu:
      acc = jnp.maximum(acc, 0.0)
    o_ref[...] = acc

  return pl.pallas_call(
      body,
      out_shape=jax.ShapeDtypeStruct((n, d), jnp.float32),
  )(sums, cnts, x, w_l, b_l.reshape(1, d), w_r)


def _split(ids2, n0, n1, nch_max):
  t0 = _NS * n0 * _CHUNK
  p0 = ids2[:, :t0].reshape(2, _NS, n0, _CHUNK).transpose(1, 2, 0, 3)
  p1 = ids2[:, t0:].reshape(2, _NS, n1, _CHUNK).transpose(1, 2, 0, 3)
  z0 = jnp.zeros((_NS, nch_max - n0, 2, _CHUNK), jnp.int32)
  z1 = jnp.zeros((_NS, nch_max - n1, 2, _CHUNK), jnp.int32)
  return jnp.concatenate([
      jnp.concatenate([p0, z0], axis=1),
      jnp.concatenate([p1, z1], axis=1)], axis=0)


def kernel(x, edge_index, W_l1, b_l1, W_r1, W_l2, b_l2, W_r2):
  n, d = x.shape
  e = edge_index.shape[1]
  tot = -(-e // (_CHUNK * _NS))
  n0 = max(4, round(tot * _FRAC0))
  n1 = tot - n0
  nch_max = max(n0, n1)
  pad = _NS * (n0 + n1) * _CHUNK - e

  src = jnp.concatenate(
      [edge_index[0].astype(jnp.int32), jnp.zeros((pad,), jnp.int32)])
  dst = jnp.concatenate(
      [edge_index[1].astype(jnp.int32), jnp.full((pad,), n, jnp.int32)])
  ids = _split(jnp.stack([src, dst]), n0, n1, nch_max)

  agg_c = _make_agg(n, d, n0, n1, True)
  agg = _make_agg(n, d, n0, n1, False)

  sums1, cnts = agg_c(x, ids)
  h = _dense_layer(sums1, cnts, x, W_l1, b_l1, W_r1, True)
  (sums2,) = agg(h, ids)
  return _dense_layer(sums2, cnts, h, W_l2, b_l2, W_r2, False)


# speedup vs baseline: 1.0101x; 1.0101x over previous
import functools

import jax
import jax.numpy as jnp
from jax import lax
from jax.experimental import pallas as pl
from jax.experimental.pallas import tpu as pltpu
from jax.experimental.pallas import tpu_sc as plsc

_NC = 2
_NS = 16
_NW = _NC * _NS
_CHUNK = 120
_LANES = 16
_FRAC0 = 0.57


def _node_pad(n_nodes: int) -> int:
  return -(-(n_nodes + 8) // (_NS * 8)) * (_NS * 8)


def _make_agg(n_nodes: int, d: int, n0: int, n1: int, with_counts: bool):
  assert min(n0, n1) >= 4
  n_pad = _node_pad(n_nodes)
  rpt = n_pad // _NS

  mesh = plsc.VectorSubcoreMesh(core_axis_name="c", subcore_axis_name="s")

  out_type = [jax.ShapeDtypeStruct((_NC * n_pad, d), jnp.float32)]
  if with_counts:
    out_type.append(jax.ShapeDtypeStruct((_NC * n_pad, d), jnp.float32))
  scratch = [
      pltpu.VMEM((4, 2, _CHUNK), jnp.int32),
      pltpu.VMEM((3, _CHUNK, d), jnp.float32),
      pltpu.VMEM_SHARED((n_pad, d), jnp.float32),
      pltpu.SemaphoreType.DMA,
      pltpu.SemaphoreType.DMA,
      pltpu.SemaphoreType.DMA,
  ]

  @functools.partial(pl.kernel, mesh=mesh, out_type=out_type,
                     scratch_types=scratch)
  def agg(table_hbm, ids_hbm, *refs):
    if with_counts:
      (out_hbm, cnt_hbm, idsw, rows_v, acc_sh, gsem, ssem, isem) = refs
    else:
      (out_hbm, idsw, rows_v, acc_sh, gsem, ssem, isem) = refs
      cnt_hbm = None

    cid = lax.axis_index("c")
    sid = lax.axis_index("s")
    w = cid * _NS + sid
    nc = jnp.where(cid == 0, n0, n1)
    r0 = pl.multiple_of(sid * rpt, 8)
    ro = pl.multiple_of(cid * n_pad + sid * rpt, 8)

    def fill_slot(slot, val):
      def _f(i, _):
        rows_v[slot, i // (d // _LANES),
               pl.ds((i % (d // _LANES)) * _LANES, _LANES)] = (
                   jnp.full((_LANES,), val, jnp.float32))
        return 0
      lax.fori_loop(0, _CHUNK * (d // _LANES), _f, 0)

    def zero_acc_slice(slot):
      done = 0
      while done < rpt:
        step = min(_CHUNK, rpt - done)
        pltpu.sync_copy(rows_v.at[slot, pl.ds(0, step)],
                        acc_sh.at[pl.ds(r0 + done, step)])
        done += step

    def drain_gather():
      pltpu.make_async_copy(table_hbm.at[pl.ds(0, _CHUNK)], rows_v.at[0],
                            gsem).wait()

    def drain_scatter():
      pltpu.make_async_copy(rows_v.at[0], acc_sh.at[pl.ds(0, _CHUNK)],
                            ssem).wait()

    def drain_ids():
      pltpu.make_async_copy(ids_hbm.at[w, 0], idsw.at[0], isem).wait()

    def sync_ids(c, slot):
      pltpu.sync_copy(ids_hbm.at[w, c], idsw.at[slot])

    def issue_ids(c, slot):
      pltpu.async_copy(ids_hbm.at[w, c], idsw.at[slot], isem)

    def issue_gather(slot_ids, slot_rows):
      pltpu.async_copy(table_hbm.at[idsw.at[slot_ids, 0]],
                       rows_v.at[slot_rows], gsem)

    def issue_scatter(slot_ids, slot_rows):
      pltpu.async_copy(rows_v.at[slot_rows], acc_sh.at[idsw.at[slot_ids, 1]],
                       ssem, add=True)

    def counts_pass():
      sync_ids(0, 0)
      issue_ids(1, 1)
      issue_ids(2, 2)
      issue_ids(3, 3)

      def _b(c, _):
        @pl.when(c >= 2)
        def _():
          drain_scatter()
          @pl.when(c + 2 < nc)
          def _():
            issue_ids(c + 2, lax.rem(c + 2, 4))
        @pl.when(c >= 1)
        def _():
          drain_ids()
        issue_scatter(lax.rem(c, 4), 0)
        return 0
      lax.fori_loop(0, nc, _b, 0)
      drain_scatter()
      drain_scatter()

    def feature_pass():
      sync_ids(0, 0)
      issue_gather(0, 0)
      issue_ids(1, 1)
      drain_ids()
      issue_gather(1, 1)
      issue_ids(2, 2)
      issue_ids(3, 3)

      def _b(c, _):
        drain_gather()
        @pl.when(c >= 1)
        def _():
          drain_scatter()
          @pl.when(c + 3 < nc)
          def _():
            issue_ids(c + 3, lax.rem(c + 3, 4))
        @pl.when(c + 2 < nc)
        def _():
          drain_ids()
          issue_gather(lax.rem(c + 2, 4), lax.rem(c + 2, 3))
        issue_scatter(lax.rem(c, 4), lax.rem(c, 3))
        return 0
      lax.fori_loop(0, nc, _b, 0)
      drain_scatter()

    def copy_out(dst_hbm_ref):
      pltpu.sync_copy(acc_sh.at[pl.ds(r0, rpt)], dst_hbm_ref.at[pl.ds(ro, rpt)])

    if with_counts:
      fill_slot(0, 1.0)
      fill_slot(1, 0.0)
      zero_acc_slice(1)
      plsc.subcore_barrier()
      counts_pass()
      plsc.subcore_barrier()
      copy_out(cnt_hbm)
      zero_acc_slice(1)
    else:
      fill_slot(1, 0.0)
      zero_acc_slice(1)
    plsc.subcore_barrier()
    feature_pass()
    plsc.subcore_barrier()
    copy_out(out_hbm)

  return agg


def _dense_layer(sums, cnts, x, w_l, b_l, w_r, apply_relu):
  n, d = x.shape
  n_pad = _node_pad(n)

  def body(s_ref, c_ref, x_ref, wl_ref, bl_ref, wr_ref, o_ref):
    s = s_ref[...]
    s = s[:n] + s[n_pad:n_pad + n]
    c = c_ref[...]
    c = c[:n, 0:1] + c[n_pad:n_pad + n, 0:1]
    mean = s * (1.0 / jnp.maximum(c, 1.0))
    acc = jnp.dot(mean, wl_ref[...], preferred_element_type=jnp.float32)
    acc = acc + jnp.dot(x_ref[...], wr_ref[...],
                        preferred_element_type=jnp.float32)
    acc = acc + bl_ref[...]
    if apply_relu:
      acc = jnp.maximum(acc, 0.0)
    o_ref[...] = acc

  return pl.pallas_call(
      body,
      out_shape=jax.ShapeDtypeStruct((n, d), jnp.float32),
  )(sums, cnts, x, w_l, b_l.reshape(1, d), w_r)


def _split(ids2, n0, n1, nch_max):
  t0 = _NS * n0 * _CHUNK
  p0 = ids2[:, :t0].reshape(2, _NS, n0, _CHUNK).transpose(1, 2, 0, 3)
  p1 = ids2[:, t0:].reshape(2, _NS, n1, _CHUNK).transpose(1, 2, 0, 3)
  z0 = jnp.zeros((_NS, nch_max - n0, 2, _CHUNK), jnp.int32)
  z1 = jnp.zeros((_NS, nch_max - n1, 2, _CHUNK), jnp.int32)
  return jnp.concatenate([
      jnp.concatenate([p0, z0], axis=1),
      jnp.concatenate([p1, z1], axis=1)], axis=0)


def kernel(x, edge_index, W_l1, b_l1, W_r1, W_l2, b_l2, W_r2):
  n, d = x.shape
  e = edge_index.shape[1]
  tot = -(-e // (_CHUNK * _NS))
  n0 = max(4, round(tot * _FRAC0))
  n1 = tot - n0
  nch_max = max(n0, n1)
  pad = _NS * (n0 + n1) * _CHUNK - e

  src = jnp.concatenate(
      [edge_index[0].astype(jnp.int32), jnp.zeros((pad,), jnp.int32)])
  dst = jnp.concatenate(
      [edge_index[1].astype(jnp.int32), jnp.full((pad,), n, jnp.int32)])
  ids = _split(jnp.stack([src, dst]), n0, n1, nch_max)

  agg_c = _make_agg(n, d, n0, n1, True)
  agg = _make_agg(n, d, n0, n1, False)

  sums1, cnts = agg_c(x, ids)
  h = _dense_layer(sums1, cnts, x, W_l1, b_l1, W_r1, True)
  (sums2,) = agg(h, ids)
  return _dense_layer(sums2, cnts, h, W_l2, b_l2, W_r2, False)


# speedup vs baseline: 1.0504x; 1.0399x over previous
import functools

import jax
import jax.numpy as jnp
from jax import lax
from jax.experimental import pallas as pl
from jax.experimental.pallas import tpu as pltpu
from jax.experimental.pallas import tpu_sc as plsc

_NC = 2
_NS = 16
_NW = _NC * _NS
_CHUNK = 120
_LANES = 16
_FRAC0 = 0.57


def _node_pad(n_nodes: int) -> int:
  return -(-(n_nodes + 8) // (_NS * 8)) * (_NS * 8)


def _make_agg(n_nodes: int, d: int, n0: int, n1: int, with_counts: bool):
  assert min(n0, n1) >= 4
  n_pad = _node_pad(n_nodes)
  rpt = n_pad // _NS

  mesh = plsc.VectorSubcoreMesh(core_axis_name="c", subcore_axis_name="s")

  out_type = [jax.ShapeDtypeStruct((_NC * n_pad, d), jnp.float32)]
  if with_counts:
    out_type.append(jax.ShapeDtypeStruct((_NC * n_pad, d), jnp.float32))
  scratch = [
      pltpu.VMEM((3, _CHUNK), jnp.int32),
      pltpu.VMEM((4, _CHUNK), jnp.int32),
      pltpu.VMEM((3, _CHUNK, d), jnp.float32),
      pltpu.VMEM_SHARED((n_pad, d), jnp.float32),
      pltpu.SemaphoreType.DMA,
      pltpu.SemaphoreType.DMA,
      pltpu.SemaphoreType.DMA,
      pltpu.SemaphoreType.DMA,
  ]

  @functools.partial(pl.kernel, mesh=mesh, out_type=out_type,
                     scratch_types=scratch)
  def agg(table_hbm, src_hbm, dst_hbm, *refs):
    if with_counts:
      (out_hbm, cnt_hbm, srcw, dstw, rows_v, acc_sh,
       gsem, ssem, isem, dsem) = refs
    else:
      (out_hbm, srcw, dstw, rows_v, acc_sh, gsem, ssem, isem, dsem) = refs
      cnt_hbm = None

    cid = lax.axis_index("c")
    sid = lax.axis_index("s")
    nc = jnp.where(cid == 0, n0, n1)
    base = jnp.where(cid == 0, sid * n0, _NS * n0 + sid * n1)
    r0 = pl.multiple_of(sid * rpt, 8)
    ro = pl.multiple_of(cid * n_pad + sid * rpt, 8)

    def fill_slot(slot, val):
      def _f(i, _):
        rows_v[slot, i // (d // _LANES),
               pl.ds((i % (d // _LANES)) * _LANES, _LANES)] = (
                   jnp.full((_LANES,), val, jnp.float32))
        return 0
      lax.fori_loop(0, _CHUNK * (d // _LANES), _f, 0)

    def zero_acc_slice(slot):
      done = 0
      while done < rpt:
        step = min(_CHUNK, rpt - done)
        pltpu.sync_copy(rows_v.at[slot, pl.ds(0, step)],
                        acc_sh.at[pl.ds(r0 + done, step)])
        done += step

    def drain_gather():
      pltpu.make_async_copy(table_hbm.at[pl.ds(0, _CHUNK)], rows_v.at[0],
                            gsem).wait()

    def drain_scatter():
      pltpu.make_async_copy(rows_v.at[0], acc_sh.at[pl.ds(0, _CHUNK)],
                            ssem).wait()

    def drain_src():
      pltpu.make_async_copy(src_hbm.at[0], srcw.at[0], isem).wait()

    def drain_dst():
      pltpu.make_async_copy(dst_hbm.at[0], dstw.at[0], dsem).wait()

    def issue_src(c, slot):
      pltpu.async_copy(src_hbm.at[base + c], srcw.at[slot], isem)

    def issue_dst(c, slot):
      pltpu.async_copy(dst_hbm.at[base + c], dstw.at[slot], dsem)

    def issue_gather(slot_ids, slot_rows):
      pltpu.async_copy(table_hbm.at[srcw.at[slot_ids]], rows_v.at[slot_rows],
                       gsem)

    def issue_scatter(slot_ids, slot_rows):
      pltpu.async_copy(rows_v.at[slot_rows], acc_sh.at[dstw.at[slot_ids]],
                       ssem, add=True)

    def counts_pass():
      for k in range(4):
        issue_dst(k, k)

      def _b(c, _):
        @pl.when(c >= 2)
        def _():
          drain_scatter()
          @pl.when(c + 2 < nc)
          def _():
            issue_dst(c + 2, lax.rem(c + 2, 4))
        drain_dst()
        issue_scatter(lax.rem(c, 4), 0)
        return 0
      lax.fori_loop(0, nc, _b, 0)
      drain_scatter()
      drain_scatter()

    def feature_pass():
      pltpu.sync_copy(src_hbm.at[base], srcw.at[0])
      issue_gather(0, 0)
      issue_src(1, 1)
      for k in range(4):
        issue_dst(k, k)
      drain_src()
      issue_gather(1, 1)
      issue_src(2, 2)

      def _b(c, _):
        drain_gather()
        @pl.when(c + 3 < nc)
        def _():
          issue_src(c + 3, lax.rem(c + 3, 3))
        @pl.when(c >= 1)
        def _():
          drain_scatter()
          @pl.when(c + 3 < nc)
          def _():
            issue_dst(c + 3, lax.rem(c + 3, 4))
        @pl.when(c + 2 < nc)
        def _():
          drain_src()
          issue_gather(lax.rem(c + 2, 3), lax.rem(c + 2, 3))
        drain_dst()
        issue_scatter(lax.rem(c, 4), lax.rem(c, 3))
        return 0
      lax.fori_loop(0, nc, _b, 0)
      drain_scatter()

    def copy_out(dst_hbm_ref):
      pltpu.sync_copy(acc_sh.at[pl.ds(r0, rpt)], dst_hbm_ref.at[pl.ds(ro, rpt)])

    if with_counts:
      fill_slot(0, 1.0)
      fill_slot(1, 0.0)
      zero_acc_slice(1)
      plsc.subcore_barrier()
      counts_pass()
      plsc.subcore_barrier()
      copy_out(cnt_hbm)
      zero_acc_slice(1)
    else:
      fill_slot(1, 0.0)
      zero_acc_slice(1)
    plsc.subcore_barrier()
    feature_pass()
    plsc.subcore_barrier()
    copy_out(out_hbm)

  return agg


def _dense_layer(sums, cnts, x, w_l, b_l, w_r, apply_relu):
  n, d = x.shape
  n_pad = _node_pad(n)

  def body(s_ref, c_ref, x_ref, wl_ref, bl_ref, wr_ref, o_ref):
    s = s_ref[...]
    s = s[:n] + s[n_pad:n_pad + n]
    c = c_ref[...]
    c = c[:n, 0:1] + c[n_pad:n_pad + n, 0:1]
    mean = s * (1.0 / jnp.maximum(c, 1.0))
    acc = jnp.dot(mean, wl_ref[...], preferred_element_type=jnp.float32)
    acc = acc + jnp.dot(x_ref[...], wr_ref[...],
                        preferred_element_type=jnp.float32)
    acc = acc + bl_ref[...]
    if apply_relu:
      acc = jnp.maximum(acc, 0.0)
    o_ref[...] = acc

  return pl.pallas_call(
      body,
      out_shape=jax.ShapeDtypeStruct((n, d), jnp.float32),
  )(sums, cnts, x, w_l, b_l.reshape(1, d), w_r)


def kernel(x, edge_index, W_l1, b_l1, W_r1, W_l2, b_l2, W_r2):
  n, d = x.shape
  e = edge_index.shape[1]
  tot = -(-e // (_CHUNK * _NS))
  n0 = max(4, round(tot * _FRAC0))
  n1 = tot - n0
  nch = _NS * (n0 + n1)
  pad = nch * _CHUNK - e

  src2 = jnp.concatenate(
      [edge_index[0].astype(jnp.int32),
       jnp.zeros((pad,), jnp.int32)]).reshape(nch, _CHUNK)
  dst2 = jnp.concatenate(
      [edge_index[1].astype(jnp.int32),
       jnp.full((pad,), n, jnp.int32)]).reshape(nch, _CHUNK)

  agg_c = _make_agg(n, d, n0, n1, True)
  agg = _make_agg(n, d, n0, n1, False)

  sums1, cnts = agg_c(x, src2, dst2)
  h = _dense_layer(sums1, cnts, x, W_l1, b_l1, W_r1, True)
  (sums2,) = agg(h, src2, dst2)
  return _dense_layer(sums2, cnts, h, W_l2, b_l2, W_r2, False)
